# Initial kernel scaffold; baseline (speedup 1.0000x reference)
import functools

import jax
import jax.numpy as jnp
from jax import lax
from jax.experimental import pallas as pl
from jax.experimental.pallas import tpu as pltpu
from jax.experimental.pallas import tpu_sc as plsc

NC = 2
NS = 16
L = 16
NW = NC * NS

VOCAB = 100000
DIM = 64
BATCH = 4096
HIST = 50
B_TOTAL = BATCH * HIST
ROWS_PER_W = B_TOTAL // NW
CHUNK = 128
NCHUNK = ROWS_PER_W // CHUNK
NB = 10
NWAVE = NCHUNK // NB
INT_MIN = jnp.int32(-(2 ** 31))


def _body(x_hbm, tab_hbm, conc_hbm, out_hbm, idxraw, idx2, conc_v, hits,
          *rest):
    bufs = rest[:NB]
    sems = rest[NB:]

    wid = lax.axis_index("s") * NC + lax.axis_index("c")
    rowbase = wid * ROWS_PER_W
    chunkbase = wid * NCHUNK

    pltpu.sync_copy(x_hbm.at[pl.ds(chunkbase, NCHUNK)], idxraw)
    pltpu.sync_copy(conc_hbm, conc_v)

    cvecs = [conc_v[pl.ds(c * L, L)] for c in range(DIM // L)]

    def clamp_chunk(s, carry):
        acc = jnp.zeros((L,), jnp.int32)
        for g in range(CHUNK // L):
            v = idxraw[s, pl.ds(g * L, L)]
            m = v == VOCAB
            idx2[s, pl.ds(g * L, L)] = jnp.where(m, 0, v)
            acc = acc + jnp.where(m, 1, 0)
        hits[s] = jnp.sum(acc)
        return carry

    lax.fori_loop(0, NCHUNK, clamp_chunk, 0)

    def patch_chunk(s, buf):
        def per_group(g, carry):
            v = idxraw[s, pl.ds(g * L, L)]
            m = v == VOCAB
            hg = jnp.sum(jnp.where(m, 1, 0))

            @pl.when(hg > 0)
            def _group():
                def per_row(r, c2):
                    lanes = lax.iota(jnp.int32, L)
                    iv = jnp.max(jnp.where(lanes == r, v, INT_MIN))

                    @pl.when(iv == VOCAB)
                    def _fix():
                        row = g * L + r
                        for c in range(DIM // L):
                            buf[row, pl.ds(c * L, L)] = cvecs[c]

                    return c2

                lax.fori_loop(0, L, per_row, 0)

            return carry

        lax.fori_loop(0, CHUNK // L, per_group, 0)

    def wave(w, carry):
        handles = []
        for b in range(NB):
            s = w * NB + b

            @pl.when(w > 0)
            def _drain(b=b, s=s):
                pltpu.make_async_copy(
                    bufs[b], out_hbm.at[pl.ds(rowbase + s * CHUNK, CHUNK)],
                    sems[b]).wait()

            handles.append(
                pltpu.async_copy(tab_hbm.at[idx2.at[s]], bufs[b], sems[b]))

        for b in range(NB):
            s = w * NB + b
            handles[b].wait()

            @pl.when(hits[s] > 0)
            def _patch(b=b, s=s):
                patch_chunk(s, bufs[b])

            pltpu.async_copy(
                bufs[b], out_hbm.at[pl.ds(rowbase + s * CHUNK, CHUNK)],
                sems[b])
        return carry

    lax.fori_loop(0, NWAVE, wave, 0)

    for b in range(NB):
        s = (NWAVE - 1) * NB + b
        pltpu.make_async_copy(
            bufs[b], out_hbm.at[pl.ds(rowbase + s * CHUNK, CHUNK)],
            sems[b]).wait()


@jax.jit
def _lookup(x2d, table, conc1d):
    scratch = [
        pltpu.VMEM((NCHUNK, CHUNK), jnp.int32),
        pltpu.VMEM((NCHUNK, CHUNK), jnp.int32),
        pltpu.VMEM((DIM,), jnp.float32),
        pltpu.SMEM((NCHUNK,), jnp.int32),
    ]
    scratch += [pltpu.VMEM((CHUNK, DIM), jnp.float32) for _ in range(NB)]
    scratch += [pltpu.SemaphoreType.DMA for _ in range(NB)]
    run = pl.kernel(
        _body,
        out_type=jax.ShapeDtypeStruct((B_TOTAL, DIM), jnp.float32),
        mesh=plsc.VectorSubcoreMesh(core_axis_name="c", subcore_axis_name="s"),
        scratch_types=scratch,
    )
    return run(x2d, table, conc1d)


def kernel(x, embed_weight, concepts):
    x2d = x.reshape(B_TOTAL // CHUNK, CHUNK).astype(jnp.int32)
    out = _lookup(x2d, embed_weight, concepts.reshape(DIM))
    return out.reshape(BATCH, HIST, DIM)


# speedup vs baseline: 4.6098x; 4.6098x over previous
import functools

import jax
import jax.numpy as jnp
import numpy as np
from jax import lax
from jax.experimental import pallas as pl
from jax.experimental.pallas import tpu as pltpu
from jax.experimental.pallas import tpu_sc as plsc

NC = 2
NS = 16
L = 16
NW = NC * NS

VOCAB = 100000
DIM = 64
BATCH = 4096
HIST = 50
B_TOTAL = BATCH * HIST
ROWS_PER_W = B_TOTAL // NW
CHUNK = 128
NCHUNK = ROWS_PER_W // CHUNK
NB = 10
NWAVE = NCHUNK // NB
INT_MIN = np.int32(-(2 ** 31))


def _xlane_gather(v, idx):
    dnums = lax.GatherDimensionNumbers(
        offset_dims=(), collapsed_slice_dims=(0,), start_index_map=(0,))
    return lax.gather(
        v, idx[:, None], dnums, (1,),
        mode=lax.GatherScatterMode.PROMISE_IN_BOUNDS)


def _body(x_hbm, tab_hbm, conc_hbm, out_hbm, idxraw, idx2, conc_v, hits,
          *rest):
    bufs = rest[:NB]
    sems = rest[NB:]
    lanes = lax.iota(jnp.int32, L)

    wid = lax.axis_index("s") * NC + lax.axis_index("c")
    rowbase = wid * ROWS_PER_W

    pltpu.sync_copy(x_hbm.at[pl.ds(rowbase, ROWS_PER_W)], idxraw)
    pltpu.sync_copy(conc_hbm, conc_v)

    cvecs = [conc_v[pl.ds(c * L, L)] for c in range(DIM // L)]

    def clamp_chunk(s, carry):
        acc = jnp.zeros((L,), jnp.int32)
        for g in range(CHUNK // L):
            v = idxraw[pl.ds(s * CHUNK + g * L, L)]
            m = v == VOCAB
            idx2[s, pl.ds(g * L, L)] = jnp.where(m, 0, v)
            acc = acc | jnp.where(m, 1, 0)
        for d in (8, 4, 2, 1):
            acc = acc | _xlane_gather(acc, lanes ^ d)
        hits[pl.ds(s * L, L)] = acc
        return carry

    lax.fori_loop(0, NCHUNK, clamp_chunk, 0)

    def patch_chunk(s, buf):
        def per_group(g, carry):
            v = idxraw[pl.ds(s * CHUNK + g * L, L)]
            for r in range(L):
                @pl.when(v[r] == VOCAB)
                def _fix(r=r):
                    for c in range(DIM // L):
                        buf[g * L + r, pl.ds(c * L, L)] = cvecs[c]
            return carry

        lax.fori_loop(0, CHUNK // L, per_group, 0)

    def wave(w, carry):
        handles = []
        for b in range(NB):
            s = w * NB + b

            @pl.when(w > 0)
            def _drain(b=b, s=s):
                pltpu.make_async_copy(
                    bufs[b], out_hbm.at[pl.ds(rowbase + s * CHUNK, CHUNK)],
                    sems[b]).wait()

            handles.append(
                pltpu.async_copy(tab_hbm.at[idx2.at[s]], bufs[b], sems[b]))

        for b in range(NB):
            s = w * NB + b
            handles[b].wait()

            hv = hits[pl.ds(s * L, L)]

            @pl.when(hv[0] > 0)
            def _patch(b=b, s=s):
                patch_chunk(s, bufs[b])

            pltpu.async_copy(
                bufs[b], out_hbm.at[pl.ds(rowbase + s * CHUNK, CHUNK)],
                sems[b])
        return carry

    lax.fori_loop(0, NWAVE, wave, 0)

    for b in range(NB):
        s = (NWAVE - 1) * NB + b
        pltpu.make_async_copy(
            bufs[b], out_hbm.at[pl.ds(rowbase + s * CHUNK, CHUNK)],
            sems[b]).wait()


@jax.jit
def _lookup(x2d, table, conc1d):
    scratch = [
        pltpu.VMEM((ROWS_PER_W,), jnp.int32),
        pltpu.VMEM((NCHUNK, CHUNK), jnp.int32),
        pltpu.VMEM((DIM,), jnp.float32),
        pltpu.VMEM((NCHUNK * L,), jnp.int32),
    ]
    scratch += [pltpu.VMEM((CHUNK, DIM), jnp.float32) for _ in range(NB)]
    scratch += [pltpu.SemaphoreType.DMA for _ in range(NB)]
    run = pl.kernel(
        _body,
        out_type=jax.ShapeDtypeStruct((B_TOTAL, DIM), jnp.float32),
        mesh=plsc.VectorSubcoreMesh(core_axis_name="c", subcore_axis_name="s"),
        scratch_types=scratch,
        compiler_params=pltpu.CompilerParams(use_tc_tiling_on_sc=False),
    )
    return run(x2d, table, conc1d)


def kernel(x, embed_weight, concepts):
    x1d = x.reshape(B_TOTAL).astype(jnp.int32)
    out = _lookup(x1d, embed_weight, concepts.reshape(DIM))
    return out.reshape(BATCH, HIST, DIM)
